# baseline (speedup 1.0000x reference)
import functools

import jax
import jax.numpy as jnp
from jax import lax
from jax.experimental import pallas as pl
from jax.experimental.pallas import tpu as pltpu
from jax.experimental.pallas import tpu_sc as plsc

NUM_ITEM = 1000000
HIDDEN = 64
BATCH = 16384
HIST = 20

_L = 16
_NC = 2
_NS = 16
_NW = _NC * _NS
_B = BATCH * HIST
_BPW = _B // _NW
_CH = 128
_NCHUNK = _BPW // _CH


def _soft_threshold_chunk(e_v, s_v, buf):
    def row_body(r, carry):
        for j in range(HIDDEN // _L):
            sl = pl.ds(j * _L, _L)
            v = e_v[buf, r, sl]
            t = s_v[buf, r, sl]
            sig = 1.0 / (1.0 + jnp.exp(-t))
            e_v[buf, r, sl] = v - jnp.minimum(jnp.maximum(v, -sig), sig)
        return carry

    lax.fori_loop(0, _CH, row_body, 0, unroll=False)


@functools.partial(
    pl.kernel,
    out_type=jax.ShapeDtypeStruct((_B, HIDDEN), jnp.float32),
    mesh=plsc.VectorSubcoreMesh(core_axis_name="c", subcore_axis_name="s"),
    compiler_params=pltpu.CompilerParams(use_tc_tiling_on_sc=False),
    scratch_types=[
        pltpu.VMEM((2, _CH), jnp.int32),
        pltpu.VMEM((2, _CH, HIDDEN), jnp.float32),
        pltpu.VMEM((2, _CH, HIDDEN), jnp.float32),
        pltpu.SemaphoreType.DMA,
        pltpu.SemaphoreType.DMA,
        pltpu.SemaphoreType.DMA,
    ],
)
def _sc_lookup(idx_hbm, emb_hbm, s_hbm, out_hbm, idx_v, e_v, s_v, sem_i, sem_e, sem_s):
    wid = lax.axis_index("s") * _NC + lax.axis_index("c")
    base = wid * _BPW

    pltpu.sync_copy(idx_hbm.at[pl.ds(base, _CH)], idx_v.at[0])
    pltpu.async_copy(emb_hbm.at[idx_v.at[0]], e_v.at[0], sem_e)
    pltpu.async_copy(s_hbm.at[idx_v.at[0]], s_v.at[0], sem_s)

    def chunk_body(c, carry):
        buf = lax.rem(c, 2)
        nbuf = lax.rem(c + 1, 2)

        @pl.when(c + 1 < _NCHUNK)
        def _():
            off_n = base + (c + 1) * _CH
            pltpu.async_copy(idx_hbm.at[pl.ds(off_n, _CH)], idx_v.at[nbuf], sem_i).wait()
            pltpu.async_copy(emb_hbm.at[idx_v.at[nbuf]], e_v.at[nbuf], sem_e)
            pltpu.async_copy(s_hbm.at[idx_v.at[nbuf]], s_v.at[nbuf], sem_s)

        pltpu.make_async_copy(emb_hbm.at[idx_v.at[buf]], e_v.at[buf], sem_e).wait()
        pltpu.make_async_copy(s_hbm.at[idx_v.at[buf]], s_v.at[buf], sem_s).wait()

        _soft_threshold_chunk(e_v, s_v, buf)
        off = base + c * _CH
        pltpu.sync_copy(e_v.at[buf], out_hbm.at[pl.ds(off, _CH)])
        return carry

    lax.fori_loop(0, _NCHUNK, chunk_body, 0, unroll=False)


def kernel(x, emb_weight, s):
    idx = x.T.reshape(-1).astype(jnp.int32)
    out = _sc_lookup(idx, emb_weight, s)
    return out.reshape(HIST, BATCH, HIDDEN).transpose(1, 0, 2)


# speedup vs baseline: 1.5785x; 1.5785x over previous
import functools

import jax
import jax.numpy as jnp
from jax import lax
from jax.experimental import pallas as pl
from jax.experimental.pallas import tpu as pltpu
from jax.experimental.pallas import tpu_sc as plsc

NUM_ITEM = 1000000
HIDDEN = 64
BATCH = 16384
HIST = 20

_L = 16
_NC = 2
_NS = 16
_NW = _NC * _NS
_B = BATCH * HIST
_BPW = _B // _NW
_CH = 128
_NCHUNK = _BPW // _CH


def _soft_threshold_chunk(e_v, s_v):
    def row_body(r, carry):
        for j in range(HIDDEN // _L):
            sl = pl.ds(j * _L, _L)
            v = e_v[r, sl]
            t = s_v[r, sl]
            sig = 1.0 / (1.0 + jnp.exp(-t))
            e_v[r, sl] = v - jnp.minimum(jnp.maximum(v, -sig), sig)
        return carry

    lax.fori_loop(0, _CH, row_body, 0, unroll=False)


@functools.partial(
    pl.kernel,
    out_type=jax.ShapeDtypeStruct((_B, HIDDEN), jnp.float32),
    mesh=plsc.VectorSubcoreMesh(core_axis_name="c", subcore_axis_name="s"),
    compiler_params=pltpu.CompilerParams(use_tc_tiling_on_sc=False),
    scratch_types=[
        pltpu.VMEM((_CH,), jnp.int32),
        pltpu.VMEM((_CH,), jnp.int32),
        pltpu.VMEM((_CH, HIDDEN), jnp.float32),
        pltpu.VMEM((_CH, HIDDEN), jnp.float32),
        pltpu.VMEM((_CH, HIDDEN), jnp.float32),
        pltpu.VMEM((_CH, HIDDEN), jnp.float32),
        pltpu.SemaphoreType.DMA,
        pltpu.SemaphoreType.DMA,
        pltpu.SemaphoreType.DMA,
        pltpu.SemaphoreType.DMA,
        pltpu.SemaphoreType.DMA,
    ],
)
def _sc_lookup(idx_hbm, emb_hbm, s_hbm, out_hbm,
               idx0, idx1, e0, e1, s0, s1,
               sem_i, sem_e0, sem_e1, sem_s0, sem_s1):
    wid = lax.axis_index("s") * _NC + lax.axis_index("c")
    base = wid * _BPW
    idx_b = (idx0, idx1)
    e_b = (e0, e1)
    s_b = (s0, s1)
    sem_e = (sem_e0, sem_e1)
    sem_s = (sem_s0, sem_s1)

    pltpu.sync_copy(idx_hbm.at[pl.ds(base, _CH)], idx0)
    pltpu.async_copy(emb_hbm.at[idx0], e0, sem_e0)
    pltpu.async_copy(s_hbm.at[idx0], s0, sem_s0)
    pltpu.async_copy(idx_hbm.at[pl.ds(base + _CH, _CH)], idx1, sem_i).wait()
    pltpu.async_copy(emb_hbm.at[idx1], e1, sem_e1)
    pltpu.async_copy(s_hbm.at[idx1], s1, sem_s1)

    def pair_body(g, carry):
        for b in range(2):
            c = g * 2 + b
            pltpu.make_async_copy(emb_hbm.at[idx_b[b]], e_b[b], sem_e[b]).wait()
            pltpu.make_async_copy(s_hbm.at[idx_b[b]], s_b[b], sem_s[b]).wait()
            _soft_threshold_chunk(e_b[b], s_b[b])
            pltpu.sync_copy(e_b[b], out_hbm.at[pl.ds(base + c * _CH, _CH)])
            @pl.when(c + 2 < _NCHUNK)
            def _():
                off_n = base + (c + 2) * _CH
                pltpu.async_copy(idx_hbm.at[pl.ds(off_n, _CH)], idx_b[b], sem_i).wait()
                pltpu.async_copy(emb_hbm.at[idx_b[b]], e_b[b], sem_e[b])
                pltpu.async_copy(s_hbm.at[idx_b[b]], s_b[b], sem_s[b])
        return carry

    lax.fori_loop(0, _NCHUNK // 2, pair_body, 0, unroll=False)


def kernel(x, emb_weight, s):
    idx = x.T.reshape(-1).astype(jnp.int32)
    out = _sc_lookup(idx, emb_weight, s)
    return out.reshape(HIST, BATCH, HIDDEN).transpose(1, 0, 2)
